# baseline (speedup 1.0000x reference)
import functools

import jax
import jax.numpy as jnp
from jax import lax
from jax.experimental import pallas as pl
from jax.experimental.pallas import tpu as pltpu
from jax.experimental.pallas import tpu_sc as plsc

_ROWS, _D = 16384, 256
_NC, _NS, _L = 2, 16, 16
_NW = _NC * _NS
_RPW = _ROWS // _NW
_C = 64
_NCH = _RPW // _C
_NB = 4
_G = _C // _L

_mesh = plsc.VectorSubcoreMesh(core_axis_name="c", subcore_axis_name="s")


@functools.partial(
    pl.kernel,
    out_type=jax.ShapeDtypeStruct((_ROWS * _D,), jnp.float32),
    mesh=_mesh,
    scratch_types=[
        pltpu.VMEM((_RPW,), jnp.int32),
        pltpu.VMEM((_D,), jnp.float32),
        [pltpu.VMEM((_C * _D,), jnp.float32) for _ in range(_NB)],
        [pltpu.SemaphoreType.DMA for _ in range(_NB)],
        [pltpu.SemaphoreType.DMA for _ in range(_NB)],
    ],
    compiler_params=pltpu.CompilerParams(needs_layout_passes=False),
)
def _sc_select(base_f, mask_hbm, oov_hbm, out_f,
               mask_v, oovrow, bufs, gsems, ssems):
    wid = lax.axis_index("s") * _NC + lax.axis_index("c")
    row0 = wid * _RPW

    pltpu.sync_copy(mask_hbm.at[pl.ds(row0, _RPW)], mask_v)
    pltpu.sync_copy(oov_hbm, oovrow)
    ovecs = [oovrow[pl.ds(j * _L, _L)] for j in range(_D // _L)]
    lane = lax.iota(jnp.int32, _L)
    zv = jnp.zeros((_L,), jnp.int32)
    qvs = [jnp.full((_L,), q, jnp.int32) for q in range(_L)]

    def gsrc(t):
        return base_f.at[pl.ds((row0 + t * _C) * _D, _C * _D)]

    def sdst(t):
        return out_f.at[pl.ds((row0 + t * _C) * _D, _C * _D)]

    def fire_gather(t):
        b = t % _NB
        if t >= _NB:
            pltpu.make_async_copy(bufs[b], sdst(t - _NB), ssems[b]).wait()
        pltpu.async_copy(gsrc(t), bufs[b], gsems[b])

    def process_and_scatter(t):
        b = t % _NB
        pltpu.make_async_copy(gsrc(t), bufs[b], gsems[b]).wait()

        def group(g, carry):
            mvec = mask_v[pl.ds(t * _C + g * _L, _L)]
            for q in range(_L):
                mr = jnp.max(jnp.where(lane == qvs[q], mvec, zv))

                def fill(g=g, q=q, b=b):
                    off = (g * _L + q) * _D
                    for j in range(_D // _L):
                        bufs[b][pl.ds(off + j * _L, _L)] = ovecs[j]

                pl.when(mr == 0)(fill)
            return carry

        lax.fori_loop(0, _G, group, 0)
        pltpu.async_copy(bufs[b], sdst(t), ssems[b])

    for t in range(_NCH):
        fire_gather(t)
        if t >= 1:
            process_and_scatter(t - 1)
    process_and_scatter(_NCH - 1)
    for t in range(_NCH - _NB, _NCH):
        b = t % _NB
        pltpu.make_async_copy(bufs[b], sdst(t), ssems[b]).wait()


def kernel(base_embedding, in_vocab, oov_embedding):
    base = base_embedding.astype(jnp.float32).reshape(_ROWS * _D)
    mask = in_vocab.astype(jnp.int32)
    out = _sc_select(base, mask, oov_embedding.astype(jnp.float32))
    return out.reshape(_ROWS, _D)
